# baseline (speedup 1.0000x reference)
import functools

import jax
import jax.numpy as jnp
from jax import lax
from jax.experimental import pallas as pl
from jax.experimental.pallas import tpu as pltpu
from jax.experimental.pallas import tpu_sc as plsc

N = 10000
E = 320000
F = 128
C = 40

NC = 2
NS = 16
NW = NC * NS
EPW = E // NW
CH = 80
NCHUNK = EPW // CH
NZT = 10
RPT = N // NZT
ZB = 40
NZB = RPT // ZB
DW = 16


@functools.lru_cache(maxsize=None)
def _make_sc_agg():
    mesh = plsc.VectorSubcoreMesh(core_axis_name="c", subcore_axis_name="s")

    @functools.partial(
        pl.kernel,
        mesh=mesh,
        out_type=jax.ShapeDtypeStruct((NC, N, F), jnp.float32),
        scratch_types=[
            pltpu.VMEM((EPW,), jnp.int32),
            pltpu.VMEM((EPW,), jnp.int32),
            pltpu.VMEM((CH, F), jnp.float32),
            pltpu.VMEM((CH, F), jnp.float32),
            pltpu.SemaphoreType.DMA,
            pltpu.SemaphoreType.DMA,
            pltpu.VMEM_SHARED((N, F), jnp.float32),
        ],
    )
    def _sc_agg(feat, srcg, dstg, zrows,
                agg_out,
                sidx_all, didx_all, rows_a, rows_b,
                sem_a, sem_b, aggsh):
        cid = lax.axis_index("c")
        sid = lax.axis_index("s")
        wid = cid * NS + sid
        r0 = sid * RPT

        @pl.when(sid < NZT)
        def _():
            def zstep(zi, carry):
                pltpu.sync_copy(zrows, aggsh.at[pl.ds(r0 + zi * ZB, ZB)])
                return carry
            lax.fori_loop(0, NZB, zstep, 0)

        pltpu.sync_copy(srcg.at[pl.ds(wid * EPW, EPW)], sidx_all)
        pltpu.sync_copy(dstg.at[pl.ds(wid * EPW, EPW)], didx_all)
        plsc.subcore_barrier()

        pltpu.async_copy(feat.at[sidx_all.at[pl.ds(0, CH)]], rows_a, sem_a)

        def pair(i, carry):
            c0 = 2 * i + 1
            pltpu.async_copy(feat.at[sidx_all.at[pl.ds(c0 * CH, CH)]], rows_b, sem_b)
            pltpu.make_async_copy(feat.at[sidx_all.at[pl.ds(0, CH)]], rows_a, sem_a).wait()
            pltpu.sync_copy(rows_a, aggsh.at[didx_all.at[pl.ds((c0 - 1) * CH, CH)]], add=True)
            pltpu.async_copy(feat.at[sidx_all.at[pl.ds((c0 + 1) * CH, CH)]], rows_a, sem_a)
            pltpu.make_async_copy(feat.at[sidx_all.at[pl.ds(0, CH)]], rows_b, sem_b).wait()
            pltpu.sync_copy(rows_b, aggsh.at[didx_all.at[pl.ds(c0 * CH, CH)]], add=True)
            return carry

        lax.fori_loop(0, (NCHUNK - 1) // 2, pair, 0)
        pltpu.make_async_copy(feat.at[sidx_all.at[pl.ds(0, CH)]], rows_a, sem_a).wait()
        pltpu.sync_copy(rows_a, aggsh.at[didx_all.at[pl.ds((NCHUNK - 1) * CH, CH)]], add=True)
        plsc.subcore_barrier()

        @pl.when(sid < NZT)
        def _():
            pltpu.sync_copy(aggsh.at[pl.ds(r0, RPT)], agg_out.at[cid, pl.ds(r0, RPT)])

    return _sc_agg


@functools.lru_cache(maxsize=None)
def _make_sc_deg():
    mesh = plsc.VectorSubcoreMesh(core_axis_name="c", subcore_axis_name="s")

    @functools.partial(
        pl.kernel,
        mesh=mesh,
        out_type=jax.ShapeDtypeStruct((NC, N, F), jnp.float32),
        scratch_types=[
            pltpu.VMEM((EPW,), jnp.int32),
            pltpu.VMEM((CH, F), jnp.float32),
            pltpu.SemaphoreType.DMA,
            pltpu.VMEM_SHARED((N, F), jnp.float32),
        ],
    )
    def _sc_deg(dstg, zdeg, ones,
                deg_out,
                didx_all, onesv, sem, degsh):
        cid = lax.axis_index("c")
        sid = lax.axis_index("s")
        wid = cid * NS + sid
        r0 = sid * RPT

        @pl.when(sid < NZT)
        def _():
            def zstep(zi, carry):
                pltpu.sync_copy(zdeg, degsh.at[pl.ds(r0 + zi * ZB, ZB)])
                return carry
            lax.fori_loop(0, NZB, zstep, 0)

        pltpu.sync_copy(ones, onesv)
        pltpu.sync_copy(dstg.at[pl.ds(wid * EPW, EPW)], didx_all)
        plsc.subcore_barrier()

        def group(gi, carry):
            c0 = gi * 5
            descs = [pltpu.async_copy(onesv, degsh.at[didx_all.at[pl.ds((c0 + j) * CH, CH)]],
                                      sem, add=True) for j in range(5)]
            for d in descs:
                d.wait()
            return carry

        lax.fori_loop(0, NCHUNK // 5, group, 0)
        plsc.subcore_barrier()

        @pl.when(sid < NZT)
        def _():
            pltpu.sync_copy(degsh.at[pl.ds(r0, RPT)], deg_out.at[cid, pl.ds(r0, RPT)])

    return _sc_deg


RB = 400
NBLK = N // RB


def _xr_body(feat, wr, bl, out_ref):
    out_ref[...] = (jnp.dot(feat[...], wr[...],
                            preferred_element_type=jnp.float32) + bl[...])


def _xr(feat, wr, bl):
    return pl.pallas_call(
        _xr_body,
        grid=(NBLK,),
        in_specs=[
            pl.BlockSpec((RB, F), lambda i: (i, 0)),
            pl.BlockSpec((F, F), lambda i: (0, 0)),
            pl.BlockSpec((1, F), lambda i: (0, 0)),
        ],
        out_specs=pl.BlockSpec((RB, F), lambda i: (i, 0)),
        out_shape=jax.ShapeDtypeStruct((N, F), jnp.float32),
    )(feat, wr, bl.reshape(1, F))


def _pre_stats_body(aggp, degp, xr, wl, pre_ref, stats_ref):
    agg = aggp[0] + aggp[1]
    deg = degp[0, :, 0:1] + degp[1, :, 0:1]
    recip = 1.0 / jnp.maximum(deg, 1.0)
    aggn = agg * recip
    pre = (jnp.dot(aggn, wl[...], preferred_element_type=jnp.float32)
           + xr[...])
    pre_ref[...] = pre
    s = jnp.sum(pre, axis=0, keepdims=True)
    s2 = jnp.sum(pre * pre, axis=0, keepdims=True)

    @pl.when(pl.program_id(0) == 0)
    def _():
        stats_ref[0:1, :] = s
        stats_ref[1:2, :] = s2

    @pl.when(pl.program_id(0) != 0)
    def _():
        stats_ref[0:1, :] += s
        stats_ref[1:2, :] += s2


def _pre_stats(aggp, degp, xr, wl):
    return pl.pallas_call(
        lambda a, d, xrr, wlr, po, so: _pre_stats_body(
            a, d, xrr, wlr, po, so),
        grid=(NBLK,),
        in_specs=[
            pl.BlockSpec((NC, RB, F), lambda i: (0, i, 0)),
            pl.BlockSpec((NC, RB, F), lambda i: (0, i, 0)),
            pl.BlockSpec((RB, F), lambda i: (i, 0)),
            pl.BlockSpec((F, F), lambda i: (0, 0)),
        ],
        out_specs=[
            pl.BlockSpec((RB, F), lambda i: (i, 0)),
            pl.BlockSpec((2, F), lambda i: (0, 0)),
        ],
        out_shape=[
            jax.ShapeDtypeStruct((N, F), jnp.float32),
            jax.ShapeDtypeStruct((2, F), jnp.float32),
        ],
    )(aggp, degp, xr, wl)


def _bn_relu_body(pre, stats, g, b, out_ref):
    m = stats[0:1, :] * (1.0 / N)
    var = stats[1:2, :] * (1.0 / N) - m * m
    inv = lax.rsqrt(var + 1e-5)
    h = (pre[...] - m) * inv * g[...] + b[...]
    out_ref[...] = jnp.maximum(h, 0.0)


def _bn_relu(pre, stats, g, b):
    return pl.pallas_call(
        _bn_relu_body,
        grid=(NBLK,),
        in_specs=[
            pl.BlockSpec((RB, F), lambda i: (i, 0)),
            pl.BlockSpec((2, F), lambda i: (0, 0)),
            pl.BlockSpec((1, F), lambda i: (0, 0)),
            pl.BlockSpec((1, F), lambda i: (0, 0)),
        ],
        out_specs=pl.BlockSpec((RB, F), lambda i: (i, 0)),
        out_shape=jax.ShapeDtypeStruct((N, F), jnp.float32),
    )(pre, stats, g.reshape(1, F), b.reshape(1, F))


def _final_body(pre, stats, g, b, wlin, blin, out_ref, logp_ref):
    m = stats[0:1, :] * (1.0 / N)
    var = stats[1:2, :] * (1.0 / N) - m * m
    inv = lax.rsqrt(var + 1e-5)
    h = (pre[...] - m) * inv * g[...] + b[...]
    h = jnp.maximum(h, 0.0)
    o = jnp.dot(h, wlin[...], preferred_element_type=jnp.float32) + blin[...]
    o = jnp.maximum(o, 0.0)
    mx = jnp.max(o, axis=1, keepdims=True)
    lse = jnp.log(jnp.sum(jnp.exp(o - mx), axis=1, keepdims=True)) + mx
    out_ref[...] = o
    logp_ref[...] = o - lse


def _final(pre, stats, g, b, wlin, blin):
    return pl.pallas_call(
        _final_body,
        grid=(NBLK,),
        in_specs=[
            pl.BlockSpec((RB, F), lambda i: (i, 0)),
            pl.BlockSpec((2, F), lambda i: (0, 0)),
            pl.BlockSpec((1, F), lambda i: (0, 0)),
            pl.BlockSpec((1, F), lambda i: (0, 0)),
            pl.BlockSpec((F, C), lambda i: (0, 0)),
            pl.BlockSpec((1, C), lambda i: (0, 0)),
        ],
        out_specs=[
            pl.BlockSpec((RB, C), lambda i: (i, 0)),
            pl.BlockSpec((RB, C), lambda i: (i, 0)),
        ],
        out_shape=[
            jax.ShapeDtypeStruct((N, C), jnp.float32),
            jax.ShapeDtypeStruct((N, C), jnp.float32),
        ],
    )(pre, stats, g.reshape(1, F), b.reshape(1, F), wlin, blin.reshape(1, C))


def kernel(x, edge_index, batch, Wl1, bl1, Wr1, g1, b1,
           Wl2, bl2, Wr2, g2, b2, Wlin, blin):
    src = edge_index[0]
    dst = edge_index[1]
    zrows = jnp.zeros((ZB, F), jnp.float32)
    zdeg = jnp.zeros((ZB, F), jnp.float32)
    ones = jnp.ones((CH, F), jnp.float32)

    degp = _make_sc_deg()(dst, zdeg, ones)
    aggp1 = _make_sc_agg()(x, src, dst, zrows)
    xr1 = _xr(x, Wr1, bl1)
    pre1, stats1 = _pre_stats(aggp1, degp, xr1, Wl1)
    h1 = _bn_relu(pre1, stats1, g1, b1)
    aggp2 = _make_sc_agg()(h1, src, dst, zrows)
    xr2 = _xr(h1, Wr2, bl2)
    pre2, stats2 = _pre_stats(aggp2, degp, xr2, Wl2)
    out, logp = _final(pre2, stats2, g2, b2, Wlin, blin)
    return (logp, out)


# speedup vs baseline: 1.0323x; 1.0323x over previous
import functools

import jax
import jax.numpy as jnp
from jax import lax
from jax.experimental import pallas as pl
from jax.experimental.pallas import tpu as pltpu
from jax.experimental.pallas import tpu_sc as plsc

N = 10000
E = 320000
F = 128
C = 40

NC = 2
NS = 16
NW = NC * NS
EPW = E // NW
CH = 80
NCHUNK = EPW // CH
NZT = 10
RPT = N // NZT
ZB = 40
NZB = RPT // ZB
DW = 16


@functools.lru_cache(maxsize=None)
def _make_sc_agg():
    mesh = plsc.VectorSubcoreMesh(core_axis_name="c", subcore_axis_name="s")

    @functools.partial(
        pl.kernel,
        mesh=mesh,
        out_type=jax.ShapeDtypeStruct((NC, N, F), jnp.float32),
        scratch_types=[
            pltpu.VMEM((EPW,), jnp.int32),
            pltpu.VMEM((EPW,), jnp.int32),
            pltpu.VMEM((CH, F), jnp.float32),
            pltpu.VMEM((CH, F), jnp.float32),
            pltpu.SemaphoreType.DMA,
            pltpu.SemaphoreType.DMA,
            pltpu.VMEM_SHARED((N, F), jnp.float32),
        ],
    )
    def _sc_agg(feat, srcg, dstg, zrows,
                agg_out,
                sidx_all, didx_all, rows_a, rows_b,
                sem_a, sem_b, aggsh):
        cid = lax.axis_index("c")
        sid = lax.axis_index("s")
        wid = cid * NS + sid
        r0 = sid * RPT

        @pl.when(sid < NZT)
        def _():
            def zstep(zi, carry):
                pltpu.sync_copy(zrows, aggsh.at[pl.ds(r0 + zi * ZB, ZB)])
                return carry
            lax.fori_loop(0, NZB, zstep, 0)

        pltpu.sync_copy(srcg.at[pl.ds(wid * EPW, EPW)], sidx_all)
        pltpu.sync_copy(dstg.at[pl.ds(wid * EPW, EPW)], didx_all)
        plsc.subcore_barrier()

        pltpu.async_copy(feat.at[sidx_all.at[pl.ds(0, CH)]], rows_a, sem_a)

        def pair(i, carry):
            c0 = 2 * i + 1
            pltpu.async_copy(feat.at[sidx_all.at[pl.ds(c0 * CH, CH)]], rows_b, sem_b)
            pltpu.make_async_copy(feat.at[sidx_all.at[pl.ds(0, CH)]], rows_a, sem_a).wait()
            pltpu.sync_copy(rows_a, aggsh.at[didx_all.at[pl.ds((c0 - 1) * CH, CH)]], add=True)
            pltpu.async_copy(feat.at[sidx_all.at[pl.ds((c0 + 1) * CH, CH)]], rows_a, sem_a)
            pltpu.make_async_copy(feat.at[sidx_all.at[pl.ds(0, CH)]], rows_b, sem_b).wait()
            pltpu.sync_copy(rows_b, aggsh.at[didx_all.at[pl.ds(c0 * CH, CH)]], add=True)
            return carry

        lax.fori_loop(0, (NCHUNK - 1) // 2, pair, 0)
        pltpu.make_async_copy(feat.at[sidx_all.at[pl.ds(0, CH)]], rows_a, sem_a).wait()
        pltpu.sync_copy(rows_a, aggsh.at[didx_all.at[pl.ds((NCHUNK - 1) * CH, CH)]], add=True)
        plsc.subcore_barrier()

        @pl.when(sid < NZT)
        def _():
            pltpu.sync_copy(aggsh.at[pl.ds(r0, RPT)], agg_out.at[cid, pl.ds(r0, RPT)])

    return _sc_agg


@functools.lru_cache(maxsize=None)
def _make_sc_agg_deg():
    mesh = plsc.VectorSubcoreMesh(core_axis_name="c", subcore_axis_name="s")

    @functools.partial(
        pl.kernel,
        mesh=mesh,
        out_type=(
            jax.ShapeDtypeStruct((NC, N, F), jnp.float32),
            jax.ShapeDtypeStruct((NC, N, F), jnp.float32),
        ),
        scratch_types=[
            pltpu.VMEM((EPW,), jnp.int32),
            pltpu.VMEM((EPW,), jnp.int32),
            pltpu.VMEM((CH, F), jnp.float32),
            pltpu.VMEM((CH, F), jnp.float32),
            pltpu.SemaphoreType.DMA,
            pltpu.SemaphoreType.DMA,
            pltpu.VMEM_SHARED((N, F), jnp.float32),
        ],
    )
    def _sc_agg_deg(feat, srcg, dstg, zrows, ones,
                    agg_out, deg_out,
                    sidx_all, didx_all, rows_a, rows_b, sem_a, sem_b, aggsh):
        cid = lax.axis_index("c")
        sid = lax.axis_index("s")
        wid = cid * NS + sid
        r0 = sid * RPT

        def _zero_acc():
            @pl.when(sid < NZT)
            def _():
                def zstep(zi, carry):
                    pltpu.sync_copy(zrows, aggsh.at[pl.ds(r0 + zi * ZB, ZB)])
                    return carry
                lax.fori_loop(0, NZB, zstep, 0)

        _zero_acc()
        pltpu.sync_copy(srcg.at[pl.ds(wid * EPW, EPW)], sidx_all)
        pltpu.sync_copy(dstg.at[pl.ds(wid * EPW, EPW)], didx_all)
        pltpu.sync_copy(ones, rows_a)
        plsc.subcore_barrier()

        def group(gi, carry):
            descs = [pltpu.async_copy(
                rows_a, aggsh.at[didx_all.at[pl.ds((gi * 5 + j) * CH, CH)]],
                sem_a, add=True) for j in range(5)]
            for d in descs:
                d.wait()
            return carry

        lax.fori_loop(0, NCHUNK // 5, group, 0)
        plsc.subcore_barrier()

        @pl.when(sid < NZT)
        def _():
            pltpu.sync_copy(aggsh.at[pl.ds(r0, RPT)], deg_out.at[cid, pl.ds(r0, RPT)])

        plsc.subcore_barrier()
        _zero_acc()
        plsc.subcore_barrier()

        pltpu.async_copy(feat.at[sidx_all.at[pl.ds(0, CH)]], rows_a, sem_a)

        def pair(i, carry):
            c0 = 2 * i + 1
            pltpu.async_copy(feat.at[sidx_all.at[pl.ds(c0 * CH, CH)]], rows_b, sem_b)
            pltpu.make_async_copy(feat.at[sidx_all.at[pl.ds(0, CH)]], rows_a, sem_a).wait()
            pltpu.sync_copy(rows_a, aggsh.at[didx_all.at[pl.ds((c0 - 1) * CH, CH)]], add=True)
            pltpu.async_copy(feat.at[sidx_all.at[pl.ds((c0 + 1) * CH, CH)]], rows_a, sem_a)
            pltpu.make_async_copy(feat.at[sidx_all.at[pl.ds(0, CH)]], rows_b, sem_b).wait()
            pltpu.sync_copy(rows_b, aggsh.at[didx_all.at[pl.ds(c0 * CH, CH)]], add=True)
            return carry

        lax.fori_loop(0, (NCHUNK - 1) // 2, pair, 0)
        pltpu.make_async_copy(feat.at[sidx_all.at[pl.ds(0, CH)]], rows_a, sem_a).wait()
        pltpu.sync_copy(rows_a, aggsh.at[didx_all.at[pl.ds((NCHUNK - 1) * CH, CH)]], add=True)
        plsc.subcore_barrier()

        @pl.when(sid < NZT)
        def _():
            pltpu.sync_copy(aggsh.at[pl.ds(r0, RPT)], agg_out.at[cid, pl.ds(r0, RPT)])

    return _sc_agg_deg


@functools.lru_cache(maxsize=None)
def _make_sc_deg():
    mesh = plsc.VectorSubcoreMesh(core_axis_name="c", subcore_axis_name="s")

    @functools.partial(
        pl.kernel,
        mesh=mesh,
        out_type=jax.ShapeDtypeStruct((NC, N, F), jnp.float32),
        scratch_types=[
            pltpu.VMEM((EPW,), jnp.int32),
            pltpu.VMEM((CH, F), jnp.float32),
            pltpu.SemaphoreType.DMA,
            pltpu.VMEM_SHARED((N, F), jnp.float32),
        ],
    )
    def _sc_deg(dstg, zdeg, ones,
                deg_out,
                didx_all, onesv, sem, degsh):
        cid = lax.axis_index("c")
        sid = lax.axis_index("s")
        wid = cid * NS + sid
        r0 = sid * RPT

        @pl.when(sid < NZT)
        def _():
            def zstep(zi, carry):
                pltpu.sync_copy(zdeg, degsh.at[pl.ds(r0 + zi * ZB, ZB)])
                return carry
            lax.fori_loop(0, NZB, zstep, 0)

        pltpu.sync_copy(ones, onesv)
        pltpu.sync_copy(dstg.at[pl.ds(wid * EPW, EPW)], didx_all)
        plsc.subcore_barrier()

        def group(gi, carry):
            c0 = gi * 5
            descs = [pltpu.async_copy(onesv, degsh.at[didx_all.at[pl.ds((c0 + j) * CH, CH)]],
                                      sem, add=True) for j in range(5)]
            for d in descs:
                d.wait()
            return carry

        lax.fori_loop(0, NCHUNK // 5, group, 0)
        plsc.subcore_barrier()

        @pl.when(sid < NZT)
        def _():
            pltpu.sync_copy(degsh.at[pl.ds(r0, RPT)], deg_out.at[cid, pl.ds(r0, RPT)])

    return _sc_deg


RB = 400
NBLK = N // RB


def _pre_stats_body(aggp, degp, feat, wl, bl, wr, pre_ref, stats_ref):
    agg = aggp[0] + aggp[1]
    deg = degp[0, :, 0:1] + degp[1, :, 0:1]
    recip = 1.0 / jnp.maximum(deg, 1.0)
    aggn = agg * recip
    pre = (jnp.dot(aggn, wl[...], preferred_element_type=jnp.float32)
           + jnp.dot(feat[...], wr[...], preferred_element_type=jnp.float32)
           + bl[...])
    pre_ref[...] = pre
    s = jnp.sum(pre, axis=0, keepdims=True)
    s2 = jnp.sum(pre * pre, axis=0, keepdims=True)

    @pl.when(pl.program_id(0) == 0)
    def _():
        stats_ref[0:1, :] = s
        stats_ref[1:2, :] = s2

    @pl.when(pl.program_id(0) != 0)
    def _():
        stats_ref[0:1, :] += s
        stats_ref[1:2, :] += s2


def _pre_stats(aggp, degp, feat, wl, bl, wr):
    return pl.pallas_call(
        lambda a, d, f, wlr, blr, wrr, po, so: _pre_stats_body(
            a, d, f, wlr, blr, wrr, po, so),
        grid=(NBLK,),
        in_specs=[
            pl.BlockSpec((NC, RB, F), lambda i: (0, i, 0)),
            pl.BlockSpec((NC, RB, F), lambda i: (0, i, 0)),
            pl.BlockSpec((RB, F), lambda i: (i, 0)),
            pl.BlockSpec((F, F), lambda i: (0, 0)),
            pl.BlockSpec((1, F), lambda i: (0, 0)),
            pl.BlockSpec((F, F), lambda i: (0, 0)),
        ],
        out_specs=[
            pl.BlockSpec((RB, F), lambda i: (i, 0)),
            pl.BlockSpec((2, F), lambda i: (0, 0)),
        ],
        out_shape=[
            jax.ShapeDtypeStruct((N, F), jnp.float32),
            jax.ShapeDtypeStruct((2, F), jnp.float32),
        ],
    )(aggp, degp, feat, wl, bl.reshape(1, F), wr)


def _bn_relu_body(pre, stats, g, b, out_ref):
    m = stats[0:1, :] * (1.0 / N)
    var = stats[1:2, :] * (1.0 / N) - m * m
    inv = lax.rsqrt(var + 1e-5)
    h = (pre[...] - m) * inv * g[...] + b[...]
    out_ref[...] = jnp.maximum(h, 0.0)


def _bn_relu(pre, stats, g, b):
    return pl.pallas_call(
        _bn_relu_body,
        grid=(NBLK,),
        in_specs=[
            pl.BlockSpec((RB, F), lambda i: (i, 0)),
            pl.BlockSpec((2, F), lambda i: (0, 0)),
            pl.BlockSpec((1, F), lambda i: (0, 0)),
            pl.BlockSpec((1, F), lambda i: (0, 0)),
        ],
        out_specs=pl.BlockSpec((RB, F), lambda i: (i, 0)),
        out_shape=jax.ShapeDtypeStruct((N, F), jnp.float32),
    )(pre, stats, g.reshape(1, F), b.reshape(1, F))


def _final_body(pre, stats, g, b, wlin, blin, out_ref, logp_ref):
    m = stats[0:1, :] * (1.0 / N)
    var = stats[1:2, :] * (1.0 / N) - m * m
    inv = lax.rsqrt(var + 1e-5)
    h = (pre[...] - m) * inv * g[...] + b[...]
    h = jnp.maximum(h, 0.0)
    o = jnp.dot(h, wlin[...], preferred_element_type=jnp.float32) + blin[...]
    o = jnp.maximum(o, 0.0)
    mx = jnp.max(o, axis=1, keepdims=True)
    lse = jnp.log(jnp.sum(jnp.exp(o - mx), axis=1, keepdims=True)) + mx
    out_ref[...] = o
    logp_ref[...] = o - lse


def _final(pre, stats, g, b, wlin, blin):
    return pl.pallas_call(
        _final_body,
        grid=(NBLK,),
        in_specs=[
            pl.BlockSpec((RB, F), lambda i: (i, 0)),
            pl.BlockSpec((2, F), lambda i: (0, 0)),
            pl.BlockSpec((1, F), lambda i: (0, 0)),
            pl.BlockSpec((1, F), lambda i: (0, 0)),
            pl.BlockSpec((F, C), lambda i: (0, 0)),
            pl.BlockSpec((1, C), lambda i: (0, 0)),
        ],
        out_specs=[
            pl.BlockSpec((RB, C), lambda i: (i, 0)),
            pl.BlockSpec((RB, C), lambda i: (i, 0)),
        ],
        out_shape=[
            jax.ShapeDtypeStruct((N, C), jnp.float32),
            jax.ShapeDtypeStruct((N, C), jnp.float32),
        ],
    )(pre, stats, g.reshape(1, F), b.reshape(1, F), wlin, blin.reshape(1, C))


def kernel(x, edge_index, batch, Wl1, bl1, Wr1, g1, b1,
           Wl2, bl2, Wr2, g2, b2, Wlin, blin):
    src = edge_index[0]
    dst = edge_index[1]
    zrows = jnp.zeros((ZB, F), jnp.float32)
    ones = jnp.ones((CH, F), jnp.float32)

    aggp1, degp = _make_sc_agg_deg()(x, src, dst, zrows, ones)
    pre1, stats1 = _pre_stats(aggp1, degp, x, Wl1, bl1, Wr1)
    h1 = _bn_relu(pre1, stats1, g1, b1)
    aggp2 = _make_sc_agg()(h1, src, dst, zrows)
    pre2, stats2 = _pre_stats(aggp2, degp, h1, Wl2, bl2, Wr2)
    out, logp = _final(pre2, stats2, g2, b2, Wlin, blin)
    return (logp, out)
